# Initial kernel scaffold; baseline (speedup 1.0000x reference)
import jax
import jax.numpy as jnp
from jax.experimental import pallas as pl


def kernel(x, edge_index, W1, b1, W2, b2):
    raise NotImplementedError("write your pallas kernel here")


# speedup vs baseline: 3.1455x; 3.1455x over previous
import functools

import jax
import jax.numpy as jnp
from jax.experimental import pallas as pl
from jax.experimental.pallas import tpu as pltpu

N_NODES = 10000
N_PAD = 10240
F_IN = 256
F_HALF = 128
HID = 512
N_CLS = 128
BLK = 1280


def _k1_body(x_ref, deg_ref, dinv_ref, ya_ref, yb_ref):
    deg = deg_ref[...]
    dinv = jnp.where(deg > 0.0, jax.lax.rsqrt(jnp.maximum(deg, 1e-12)), 0.0)
    dinv_ref[...] = dinv
    x = x_ref[...]
    ya_ref[...] = x[:, :F_HALF] * dinv
    yb_ref[...] = x[:, F_HALF:] * dinv


def _k1(x_pad, deg_pad):
    return pl.pallas_call(
        _k1_body,
        out_shape=(
            jax.ShapeDtypeStruct((N_PAD, 1), jnp.float32),
            jax.ShapeDtypeStruct((N_PAD, F_HALF), jnp.float32),
            jax.ShapeDtypeStruct((N_PAD, F_HALF), jnp.float32),
        ),
    )(x_pad, deg_pad)


def _k2_body(a1_ref, a2_ref, ya_ref, yb_ref, dinv_ref, w1a_ref, w1b_ref,
             b1_ref, w2_ref, z_ref):
    dinv = dinv_ref[...]
    u1 = (a1_ref[...] + ya_ref[...]) * dinv
    u2 = (a2_ref[...] + yb_ref[...]) * dinv
    h = (jnp.dot(u1, w1a_ref[...], preferred_element_type=jnp.float32)
         + jnp.dot(u2, w1b_ref[...], preferred_element_type=jnp.float32)
         + b1_ref[...])
    h = jnp.maximum(h, 0.0)
    t = jnp.dot(h, w2_ref[...], preferred_element_type=jnp.float32)
    z_ref[...] = t * dinv


def _k2(a1, a2, ya, yb, dinv, W1, b1, W2):
    grid = (N_PAD // BLK,)
    row_spec = lambda w: pl.BlockSpec((BLK, w), lambda i: (i, 0))
    full = lambda shape: pl.BlockSpec(shape, lambda i: (0,) * len(shape))
    return pl.pallas_call(
        _k2_body,
        grid=grid,
        in_specs=[
            row_spec(F_HALF), row_spec(F_HALF),
            row_spec(F_HALF), row_spec(F_HALF),
            row_spec(1),
            full((F_HALF, HID)), full((F_HALF, HID)),
            full((1, HID)),
            full((HID, N_CLS)),
        ],
        out_specs=row_spec(N_CLS),
        out_shape=jax.ShapeDtypeStruct((N_PAD, N_CLS), jnp.float32),
    )(a1, a2, ya, yb, dinv, W1[:F_HALF], W1[F_HALF:], b1[None, :], W2)


def _k3_body(p0_ref, p1_ref, z_ref, dinv_ref, b2_ref, o_ref):
    o = (p0_ref[...] + p1_ref[...] + z_ref[...]) * dinv_ref[...] + b2_ref[...]
    m = jnp.max(o, axis=1, keepdims=True)
    e = jnp.exp(o - m)
    lse = jnp.log(jnp.sum(e, axis=1, keepdims=True))
    o_ref[...] = o - m - lse


def _k3(p0, p1, z, dinv, b2):
    grid = (N_PAD // BLK,)
    row_spec = lambda w: pl.BlockSpec((BLK, w), lambda i: (i, 0))
    return pl.pallas_call(
        _k3_body,
        grid=grid,
        in_specs=[
            row_spec(N_CLS), row_spec(N_CLS), row_spec(N_CLS), row_spec(1),
            pl.BlockSpec((1, N_CLS), lambda i: (0, 0)),
        ],
        out_specs=row_spec(N_CLS),
        out_shape=jax.ShapeDtypeStruct((N_PAD, N_CLS), jnp.float32),
    )(p0, p1, z, dinv, b2[None, :])


def kernel(x, edge_index, W1, b1, W2, b2):
    src = edge_index[0]
    dst = edge_index[1]

    deg = jnp.zeros((N_NODES,), jnp.float32).at[dst].add(1.0) + 1.0
    deg_pad = jnp.pad(deg, (0, N_PAD - N_NODES))[:, None]
    x_pad = jnp.pad(x, ((0, N_PAD - N_NODES), (0, 0)))

    dinv, ya, yb = _k1(x_pad, deg_pad)

    a1 = jnp.zeros((N_PAD, F_HALF), jnp.float32).at[dst].add(ya[src])
    a2 = jnp.zeros((N_PAD, F_HALF), jnp.float32).at[dst].add(yb[src])

    z = _k2(a1, a2, ya, yb, dinv, W1, b1, W2)

    p0 = jnp.zeros((N_PAD, N_CLS), jnp.float32).at[dst].add(z[src])
    p1 = jnp.zeros((N_PAD, N_CLS), jnp.float32)

    out = _k3(p0, p1, z, dinv, b2)
    return out[:N_NODES]


# speedup vs baseline: 8.2795x; 2.6322x over previous
import functools

import jax
import jax.numpy as jnp
from jax.experimental import pallas as pl
from jax.experimental.pallas import tpu as pltpu
from jax.experimental.pallas import tpu_sc as plsc

N_NODES = 10000
N_PAD = 10240
F_IN = 256
F_HALF = 128
HID = 512
N_CLS = 128
BLK = 1280
N_TILES = 16
CHUNK = 128
DUMMY = 10008
E_PAD = 2 * N_TILES * 40 * CHUNK
ACC_GAP = 512
ACC_ALLOC = N_PAD + ACC_GAP

_SC_MESH = plsc.VectorSubcoreMesh(core_axis_name="c", subcore_axis_name="s")


def _sc_fill(ref, val):
    v = jnp.full((16,), val, jnp.float32)

    @pl.loop(0, ref.shape[0])
    def _(i):
        @pl.loop(0, ref.shape[1], step=16)
        def _(k):
            ref[i, pl.ds(k, 16)] = v


def _sc_deg(dst_idx):
    nchunks = 40
    rows = N_PAD // N_TILES

    @functools.partial(
        pl.kernel,
        out_type=jax.ShapeDtypeStruct((2 * N_PAD, F_HALF), jnp.float32),
        mesh=_SC_MESH,
        scratch_types=[
            pltpu.VMEM((nchunks, CHUNK), jnp.int32),
            pltpu.VMEM((CHUNK, F_HALF), jnp.float32),
            pltpu.VMEM_SHARED((ACC_ALLOC, F_HALF), jnp.float32),
        ],
    )
    def k(dst_hbm, out_hbm, idx_v, buf_v, acc_sh):
        cid = jax.lax.axis_index("c")
        sid = jax.lax.axis_index("s")
        base = sid * rows + jnp.where(sid >= 2, ACC_GAP, 0)
        wid = cid * N_TILES + sid

        _sc_fill(buf_v, 0.0)

        @pl.loop(0, rows, step=CHUNK)
        def _(r):
            pltpu.sync_copy(buf_v, acc_sh.at[pl.ds(base + r, CHUNK)])

        pltpu.sync_copy(dst_hbm.at[pl.ds(wid * nchunks, nchunks)], idx_v)
        _sc_fill(buf_v, 1.0)
        plsc.subcore_barrier()

        @pl.loop(0, nchunks)
        def _(j):
            pltpu.sync_copy(buf_v, acc_sh.at[idx_v.at[j]], add=True)

        plsc.subcore_barrier()
        pltpu.sync_copy(acc_sh.at[pl.ds(base, rows)],
                        out_hbm.at[pl.ds(cid * N_PAD + sid * rows, rows)])

    return k(dst_idx)


def _sc_agg(table, src_idx, dst_idx, nchunks):
    rows = N_PAD // N_TILES

    @functools.partial(
        pl.kernel,
        out_type=jax.ShapeDtypeStruct((2 * N_PAD, F_HALF), jnp.float32),
        mesh=_SC_MESH,
        scratch_types=[
            pltpu.VMEM((nchunks, CHUNK), jnp.int32),
            pltpu.VMEM((nchunks, CHUNK), jnp.int32),
            pltpu.VMEM((CHUNK, F_HALF), jnp.float32),
            pltpu.VMEM_SHARED((ACC_ALLOC, F_HALF), jnp.float32),
            pltpu.SemaphoreType.DMA,
        ],
    )
    def k(tab_hbm, src_hbm, dst_hbm, out_hbm, srcv, dstv, buf, acc_sh, sem):
        cid = jax.lax.axis_index("c")
        sid = jax.lax.axis_index("s")
        base = sid * rows + jnp.where(sid >= 2, ACC_GAP, 0)
        wid = cid * N_TILES + sid

        _sc_fill(buf, 0.0)

        @pl.loop(0, rows, step=CHUNK)
        def _(r):
            pltpu.sync_copy(buf, acc_sh.at[pl.ds(base + r, CHUNK)])

        pltpu.sync_copy(src_hbm.at[pl.ds(wid * nchunks, nchunks)], srcv)
        pltpu.sync_copy(dst_hbm.at[pl.ds(wid * nchunks, nchunks)], dstv)
        plsc.subcore_barrier()

        @pl.loop(0, nchunks)
        def _(j):
            pltpu.async_copy(tab_hbm.at[srcv.at[j]], buf, sem).wait()
            pltpu.sync_copy(buf, acc_sh.at[dstv.at[j]], add=True)

        plsc.subcore_barrier()
        pltpu.sync_copy(acc_sh.at[pl.ds(base, rows)],
                        out_hbm.at[pl.ds(cid * N_PAD + sid * rows, rows)])

    return k(table, src_idx, dst_idx)


def _k1_body(x_ref, dp0_ref, dp1_ref, dinv_ref, ya_ref, yb_ref):
    deg = 1.0 + dp0_ref[:, 0:1] + dp1_ref[:, 0:1]
    row = jax.lax.broadcasted_iota(jnp.int32, (N_PAD, 1), 0)
    dinv = jnp.where(row < N_NODES,
                     jax.lax.rsqrt(jnp.maximum(deg, 1e-12)), 0.0)
    dinv_ref[...] = dinv
    x = x_ref[...]
    ya_ref[...] = x[:, :F_HALF] * dinv
    yb_ref[...] = x[:, F_HALF:] * dinv


def _k1(x_pad, dp0, dp1):
    return pl.pallas_call(
        _k1_body,
        out_shape=(
            jax.ShapeDtypeStruct((N_PAD, 1), jnp.float32),
            jax.ShapeDtypeStruct((N_PAD, F_HALF), jnp.float32),
            jax.ShapeDtypeStruct((N_PAD, F_HALF), jnp.float32),
        ),
    )(x_pad, dp0, dp1)


def _k2_body(a1_ref, a2_ref, ya_ref, yb_ref, dinv_ref, w1a_ref, w1b_ref,
             b1_ref, w2_ref, z_ref):
    dinv = dinv_ref[...]
    u1 = (a1_ref[...] + ya_ref[...]) * dinv
    u2 = (a2_ref[...] + yb_ref[...]) * dinv
    h = (jnp.dot(u1, w1a_ref[...], preferred_element_type=jnp.float32)
         + jnp.dot(u2, w1b_ref[...], preferred_element_type=jnp.float32)
         + b1_ref[...])
    h = jnp.maximum(h, 0.0)
    t = jnp.dot(h, w2_ref[...], preferred_element_type=jnp.float32)
    z_ref[...] = t * dinv


def _k2(a1, a2, ya, yb, dinv, W1, b1, W2):
    grid = (N_PAD // BLK,)
    row_spec = lambda w: pl.BlockSpec((BLK, w), lambda i: (i, 0))
    full = lambda shape: pl.BlockSpec(shape, lambda i: (0,) * len(shape))
    return pl.pallas_call(
        _k2_body,
        grid=grid,
        in_specs=[
            row_spec(F_HALF), row_spec(F_HALF),
            row_spec(F_HALF), row_spec(F_HALF),
            row_spec(1),
            full((F_HALF, HID)), full((F_HALF, HID)),
            full((1, HID)),
            full((HID, N_CLS)),
        ],
        out_specs=row_spec(N_CLS),
        out_shape=jax.ShapeDtypeStruct((N_PAD, N_CLS), jnp.float32),
    )(a1, a2, ya, yb, dinv, W1[:F_HALF], W1[F_HALF:], b1[None, :], W2)


def _k3_body(p0_ref, p1_ref, z_ref, dinv_ref, b2_ref, o_ref):
    o = (p0_ref[...] + p1_ref[...] + z_ref[...]) * dinv_ref[...] + b2_ref[...]
    m = jnp.max(o, axis=1, keepdims=True)
    e = jnp.exp(o - m)
    lse = jnp.log(jnp.sum(e, axis=1, keepdims=True))
    o_ref[...] = o - m - lse


def _k3(p0, p1, z, dinv, b2):
    grid = (N_PAD // BLK,)
    row_spec = lambda w: pl.BlockSpec((BLK, w), lambda i: (i, 0))
    return pl.pallas_call(
        _k3_body,
        grid=grid,
        in_specs=[
            row_spec(N_CLS), row_spec(N_CLS), row_spec(N_CLS), row_spec(1),
            pl.BlockSpec((1, N_CLS), lambda i: (0, 0)),
        ],
        out_specs=row_spec(N_CLS),
        out_shape=jax.ShapeDtypeStruct((N_PAD, N_CLS), jnp.float32),
    )(p0, p1, z, dinv, b2[None, :])


def kernel(x, edge_index, W1, b1, W2, b2):
    src = edge_index[0]
    dst = edge_index[1]
    e = src.shape[0]

    pad = E_PAD - e
    src_p = jnp.concatenate([src, jnp.full((pad,), DUMMY, jnp.int32)])
    dst_p = jnp.concatenate([dst, jnp.full((pad,), DUMMY, jnp.int32)])
    dst_p = dst_p + jnp.where(dst_p >= 1280, ACC_GAP, 0).astype(jnp.int32)

    src_32 = src_p.reshape(2 * N_TILES * 40, CHUNK)
    dst_32 = dst_p.reshape(2 * N_TILES * 40, CHUNK)
    src_16 = src_p.reshape(1, N_TILES * 80, CHUNK)
    offs = jnp.array([0, N_PAD], jnp.int32).reshape(2, 1, 1)
    src_a1 = (src_16 + offs).reshape(2 * N_TILES * 80, CHUNK)
    dst_a1 = jnp.broadcast_to(dst_p.reshape(1, N_TILES * 80, CHUNK),
                              (2, N_TILES * 80, CHUNK)).reshape(-1, CHUNK)

    dp = _sc_deg(dst_32)
    x_pad = jnp.pad(x, ((0, N_PAD - N_NODES), (0, 0)))
    dinv, ya, yb = _k1(x_pad, dp[:N_PAD], dp[N_PAD:])

    y_flat = jnp.concatenate([ya, yb], axis=0)
    a = _sc_agg(y_flat, src_a1, dst_a1, 80)
    z = _k2(a[:N_PAD], a[N_PAD:], ya, yb, dinv, W1, b1, W2)

    p = _sc_agg(z, src_32, dst_32, 40)
    out = _k3(p[:N_PAD], p[N_PAD:], z, dinv, b2)
    return out[:N_NODES]


# speedup vs baseline: 9.3886x; 1.1340x over previous
import functools

import jax
import jax.numpy as jnp
from jax.experimental import pallas as pl
from jax.experimental.pallas import tpu as pltpu
from jax.experimental.pallas import tpu_sc as plsc

N_NODES = 10000
N_PAD = 10240
F_IN = 256
F_HALF = 128
HID = 512
N_CLS = 128
BLK = 1280
N_TILES = 16
CHUNK = 128
DUMMY = 10008
E_PAD = 2 * N_TILES * 40 * CHUNK
ACC_GAP = 512
ACC_ALLOC = N_PAD + ACC_GAP

_SC_MESH = plsc.VectorSubcoreMesh(core_axis_name="c", subcore_axis_name="s")


def _sc_fill(ref, val):
    v = jnp.full((16,), val, jnp.float32)

    @pl.loop(0, ref.shape[0])
    def _(i):
        @pl.loop(0, ref.shape[1], step=16)
        def _(k):
            ref[i, pl.ds(k, 16)] = v


def _sc_deg(dst_idx):
    nchunks = 40
    rows = N_PAD // N_TILES

    @functools.partial(
        pl.kernel,
        out_type=jax.ShapeDtypeStruct((2 * N_PAD, F_HALF), jnp.float32),
        mesh=_SC_MESH,
        scratch_types=[
            pltpu.VMEM((nchunks, CHUNK), jnp.int32),
            pltpu.VMEM((CHUNK, F_HALF), jnp.float32),
            pltpu.VMEM_SHARED((ACC_ALLOC, F_HALF), jnp.float32),
        ],
    )
    def k(dst_hbm, out_hbm, idx_v, buf_v, acc_sh):
        cid = jax.lax.axis_index("c")
        sid = jax.lax.axis_index("s")
        base = sid * rows + jnp.where(sid >= 2, ACC_GAP, 0)
        wid = cid * N_TILES + sid

        _sc_fill(buf_v, 0.0)

        @pl.loop(0, rows, step=CHUNK)
        def _(r):
            pltpu.sync_copy(buf_v, acc_sh.at[pl.ds(base + r, CHUNK)])

        pltpu.sync_copy(dst_hbm.at[pl.ds(wid * nchunks, nchunks)], idx_v)
        _sc_fill(buf_v, 1.0)
        plsc.subcore_barrier()

        @pl.loop(0, nchunks)
        def _(j):
            pltpu.sync_copy(buf_v, acc_sh.at[idx_v.at[j]], add=True)

        plsc.subcore_barrier()
        pltpu.sync_copy(acc_sh.at[pl.ds(base, rows)],
                        out_hbm.at[pl.ds(cid * N_PAD + sid * rows, rows)])

    return k(dst_idx)


def _sc_agg(table, src_idx, dst_idx, nchunks):
    rows = N_PAD // N_TILES

    @functools.partial(
        pl.kernel,
        out_type=jax.ShapeDtypeStruct((2 * N_PAD, F_HALF), jnp.float32),
        mesh=_SC_MESH,
        scratch_types=[
            pltpu.VMEM((4, CHUNK), jnp.int32),
            pltpu.VMEM((4, CHUNK), jnp.int32),
            pltpu.VMEM((CHUNK, F_HALF), jnp.float32),
            pltpu.VMEM((CHUNK, F_HALF), jnp.float32),
            pltpu.VMEM_SHARED((ACC_ALLOC, F_HALF), jnp.float32),
            pltpu.SemaphoreType.DMA,
            pltpu.SemaphoreType.DMA,
            pltpu.SemaphoreType.DMA,
            pltpu.SemaphoreType.DMA,
            pltpu.SemaphoreType.DMA,
            pltpu.SemaphoreType.DMA,
        ],
    )
    def k(tab_hbm, src_hbm, dst_hbm, out_hbm, srcv, dstv, buf_a, buf_b,
          acc_sh, sem_a, sem_b, si0, si1, si2, si3):
        cid = jax.lax.axis_index("c")
        sid = jax.lax.axis_index("s")
        base = sid * rows + jnp.where(sid >= 2, ACC_GAP, 0)
        wid = cid * N_TILES + sid
        ibase = wid * nchunks
        sem_i = [si0, si1, si2, si3]
        bufs = [buf_a, buf_b]
        sem_g = [sem_a, sem_b]

        def idx_load(chunk, slot):
            pltpu.async_copy(src_hbm.at[ibase + chunk], srcv.at[slot],
                             sem_i[slot])
            pltpu.async_copy(dst_hbm.at[ibase + chunk], dstv.at[slot],
                             sem_i[slot])

        def idx_wait(slot):
            pltpu.make_async_copy(src_hbm.at[0], srcv.at[slot],
                                  sem_i[slot]).wait()
            pltpu.make_async_copy(dst_hbm.at[0], dstv.at[slot],
                                  sem_i[slot]).wait()

        def gather(slot, b):
            pltpu.async_copy(tab_hbm.at[srcv.at[slot]], bufs[b], sem_g[b])

        def gather_wait(b):
            pltpu.make_async_copy(tab_hbm.at[srcv.at[0]], bufs[b],
                                  sem_g[b]).wait()

        def scatter(slot, b):
            pltpu.sync_copy(bufs[b], acc_sh.at[dstv.at[slot]], add=True)

        _sc_fill(buf_a, 0.0)

        @pl.loop(0, rows, step=CHUNK)
        def _(r):
            pltpu.sync_copy(buf_a, acc_sh.at[pl.ds(base + r, CHUNK)])

        plsc.subcore_barrier()

        for s in range(4):
            idx_load(s, s)
        idx_wait(0)
        gather(0, 0)
        idx_wait(1)
        gather(1, 1)

        @pl.loop(0, nchunks - 4, step=4)
        def _(j):
            gather_wait(0)
            scatter(0, 0)
            idx_load(j + 4, 0)
            idx_wait(2)
            gather(2, 0)

            gather_wait(1)
            scatter(1, 1)
            idx_load(j + 5, 1)
            idx_wait(3)
            gather(3, 1)

            gather_wait(0)
            scatter(2, 0)
            idx_load(j + 6, 2)
            idx_wait(0)
            gather(0, 0)

            gather_wait(1)
            scatter(3, 1)
            idx_load(j + 7, 3)
            idx_wait(1)
            gather(1, 1)

        gather_wait(0)
        scatter(0, 0)
        idx_wait(2)
        gather(2, 0)
        gather_wait(1)
        scatter(1, 1)
        idx_wait(3)
        gather(3, 1)
        gather_wait(0)
        scatter(2, 0)
        gather_wait(1)
        scatter(3, 1)

        plsc.subcore_barrier()
        pltpu.sync_copy(acc_sh.at[pl.ds(base, rows)],
                        out_hbm.at[pl.ds(cid * N_PAD + sid * rows, rows)])

    return k(table, src_idx, dst_idx)


def _k1_body(x_ref, dp0_ref, dp1_ref, dinv_ref, ya_ref, yb_ref):
    deg = 1.0 + dp0_ref[:, 0:1] + dp1_ref[:, 0:1]
    row = jax.lax.broadcasted_iota(jnp.int32, (N_PAD, 1), 0)
    dinv = jnp.where(row < N_NODES,
                     jax.lax.rsqrt(jnp.maximum(deg, 1e-12)), 0.0)
    dinv_ref[...] = dinv
    x = x_ref[...]
    ya_ref[...] = x[:, :F_HALF] * dinv
    yb_ref[...] = x[:, F_HALF:] * dinv


def _k1(x_pad, dp0, dp1):
    return pl.pallas_call(
        _k1_body,
        out_shape=(
            jax.ShapeDtypeStruct((N_PAD, 1), jnp.float32),
            jax.ShapeDtypeStruct((N_PAD, F_HALF), jnp.float32),
            jax.ShapeDtypeStruct((N_PAD, F_HALF), jnp.float32),
        ),
    )(x_pad, dp0, dp1)


def _k2_body(a1_ref, a2_ref, ya_ref, yb_ref, dinv_ref, w1a_ref, w1b_ref,
             b1_ref, w2_ref, z_ref):
    dinv = dinv_ref[...]
    u1 = (a1_ref[...] + ya_ref[...]) * dinv
    u2 = (a2_ref[...] + yb_ref[...]) * dinv
    h = (jnp.dot(u1, w1a_ref[...], preferred_element_type=jnp.float32)
         + jnp.dot(u2, w1b_ref[...], preferred_element_type=jnp.float32)
         + b1_ref[...])
    h = jnp.maximum(h, 0.0)
    t = jnp.dot(h, w2_ref[...], preferred_element_type=jnp.float32)
    z_ref[...] = t * dinv


def _k2(a1, a2, ya, yb, dinv, W1, b1, W2):
    grid = (N_PAD // BLK,)
    row_spec = lambda w: pl.BlockSpec((BLK, w), lambda i: (i, 0))
    full = lambda shape: pl.BlockSpec(shape, lambda i: (0,) * len(shape))
    return pl.pallas_call(
        _k2_body,
        grid=grid,
        in_specs=[
            row_spec(F_HALF), row_spec(F_HALF),
            row_spec(F_HALF), row_spec(F_HALF),
            row_spec(1),
            full((F_HALF, HID)), full((F_HALF, HID)),
            full((1, HID)),
            full((HID, N_CLS)),
        ],
        out_specs=row_spec(N_CLS),
        out_shape=jax.ShapeDtypeStruct((N_PAD, N_CLS), jnp.float32),
    )(a1, a2, ya, yb, dinv, W1[:F_HALF], W1[F_HALF:], b1[None, :], W2)


def _k3_body(p0_ref, p1_ref, z_ref, dinv_ref, b2_ref, o_ref):
    o = (p0_ref[...] + p1_ref[...] + z_ref[...]) * dinv_ref[...] + b2_ref[...]
    m = jnp.max(o, axis=1, keepdims=True)
    e = jnp.exp(o - m)
    lse = jnp.log(jnp.sum(e, axis=1, keepdims=True))
    o_ref[...] = o - m - lse


def _k3(p0, p1, z, dinv, b2):
    grid = (N_PAD // BLK,)
    row_spec = lambda w: pl.BlockSpec((BLK, w), lambda i: (i, 0))
    return pl.pallas_call(
        _k3_body,
        grid=grid,
        in_specs=[
            row_spec(N_CLS), row_spec(N_CLS), row_spec(N_CLS), row_spec(1),
            pl.BlockSpec((1, N_CLS), lambda i: (0, 0)),
        ],
        out_specs=row_spec(N_CLS),
        out_shape=jax.ShapeDtypeStruct((N_PAD, N_CLS), jnp.float32),
    )(p0, p1, z, dinv, b2[None, :])


def kernel(x, edge_index, W1, b1, W2, b2):
    src = edge_index[0]
    dst = edge_index[1]
    e = src.shape[0]

    pad = E_PAD - e
    src_p = jnp.concatenate([src, jnp.full((pad,), DUMMY, jnp.int32)])
    dst_p = jnp.concatenate([dst, jnp.full((pad,), DUMMY, jnp.int32)])
    dst_p = dst_p + jnp.where(dst_p >= 1280, ACC_GAP, 0).astype(jnp.int32)

    src_32 = src_p.reshape(2 * N_TILES * 40, CHUNK)
    dst_32 = dst_p.reshape(2 * N_TILES * 40, CHUNK)
    src_16 = src_p.reshape(1, N_TILES * 80, CHUNK)
    offs = jnp.array([0, N_PAD], jnp.int32).reshape(2, 1, 1)
    src_a1 = (src_16 + offs).reshape(2 * N_TILES * 80, CHUNK)
    dst_a1 = jnp.broadcast_to(dst_p.reshape(1, N_TILES * 80, CHUNK),
                              (2, N_TILES * 80, CHUNK)).reshape(-1, CHUNK)

    dp = _sc_deg(dst_32)
    x_pad = jnp.pad(x, ((0, N_PAD - N_NODES), (0, 0)))
    dinv, ya, yb = _k1(x_pad, dp[:N_PAD], dp[N_PAD:])

    y_flat = jnp.concatenate([ya, yb], axis=0)
    a = _sc_agg(y_flat, src_a1, dst_a1, 80)
    z = _k2(a[:N_PAD], a[N_PAD:], ya, yb, dinv, W1, b1, W2)

    p = _sc_agg(z, src_32, dst_32, 40)
    out = _k3(p[:N_PAD], p[N_PAD:], z, dinv, b2)
    return out[:N_NODES]
